# baseline (speedup 1.0000x reference)
import functools

import jax
import jax.numpy as jnp
from jax import lax
from jax.experimental import pallas as pl
from jax.experimental.pallas import tpu as pltpu
from jax.experimental.pallas import tpu_sc as plsc

K = 16
NEG_INF = float("-inf")
HI = lax.Precision.HIGHEST


def _gelu(v):
    return 0.5 * v * (1.0 + lax.erf(v * 0.7071067811865476))


def _topk_idx(p, iota_l):
    N = p.shape[0]
    pp = p * p
    rcol = jnp.sum(pp, axis=1, keepdims=True)
    rrow = jnp.reshape(jnp.sum(pp, axis=1), (1, N))
    m = lax.dot_general(p, p, (((1,), (1,)), ((), ())), precision=HI)
    nd = -(rcol - 2.0 * m + rrow + 1e-05)
    sels = []
    for _ in range(K + 1):
        sel = jnp.argmax(nd, axis=1).astype(jnp.int32)[:, None]
        sels.append(sel)
        nd = jnp.where(iota_l == sel, NEG_INF, nd)
    return sels


def _idx_out(sels, base, N):
    idxmat = jnp.concatenate(sels[1:], axis=1)
    return jnp.transpose(idxmat, (1, 0)) + base


def _mlp(g0_ref, f, W1, b1, W2, b2, prec):
    N, F = f.shape
    W1a = W1[:F, :]
    W1d = W1[F:, :] - W1a
    cterm = lax.dot_general(f, W1d, (((1,), (0,)), ((), ())),
                            precision=prec) + b1
    acc = None
    for k in range(K):
        g = g0_ref[0, pl.ds(k * N, N), :F]
        h1 = _gelu(lax.dot_general(g, W1a, (((1,), (0,)), ((), ())),
                                   precision=prec) + cterm)
        h2 = _gelu(lax.dot_general(h1, W2, (((1,), (0,)), ((), ())),
                                   precision=prec) + b2)
        acc = h2 if acc is None else acc + h2
    return acc * (1.0 / K)


def _body_a(pts_ref, mask_ref, idx_ref):
    pts = pts_ref[0]
    mcol = mask_ref[0]
    N = pts.shape[0]
    shift = 999.0 * (mcol == 0.0).astype(jnp.float32)
    iota_l = lax.broadcasted_iota(jnp.int32, (N, N), 1)
    sels = _topk_idx(shift + pts, iota_l)
    idx_ref[0] = _idx_out(sels, pl.program_id(0) * N, N)


def _body_b(x_ref, g0_ref, mask_ref, W01_ref, b01_ref, W02_ref, b02_ref,
            f1_ref, idx_ref):
    f = x_ref[0]
    mcol = mask_ref[0]
    N = f.shape[0]
    shift = 999.0 * (mcol == 0.0).astype(jnp.float32)
    iota_l = lax.broadcasted_iota(jnp.int32, (N, N), 1)
    f1 = _mlp(g0_ref, f, W01_ref[...], b01_ref[...], W02_ref[...],
              b02_ref[...], HI)
    f1_ref[0] = f1
    sels = _topk_idx(shift + f1, iota_l)
    idx_ref[0] = _idx_out(sels, pl.program_id(0) * N, N)


def _body_c(f1_ref, g1_ref, mask_ref, W11_ref, b11_ref, W12_ref, b12_ref,
            out_ref):
    f1 = f1_ref[0]
    mcol = mask_ref[0]
    f2 = _mlp(g1_ref, f1, W11_ref[...], b11_ref[...], W12_ref[...],
              b12_ref[...], lax.Precision.DEFAULT)
    out_ref[0] = f2 * mcol


def _sc_gather(table, idx):
    R, F = table.shape
    M = idx.shape[0]
    NW = 32
    CH = 128
    per_w = M // NW
    n_chunks = per_w // CH
    mesh = plsc.VectorSubcoreMesh(core_axis_name="c", subcore_axis_name="s")

    @functools.partial(
        pl.kernel, mesh=mesh,
        out_type=jax.ShapeDtypeStruct((M, F), jnp.float32),
        scratch_types=[
            pltpu.VMEM((CH,), jnp.int32),
            pltpu.VMEM((CH, F), jnp.float32),
            pltpu.SemaphoreType.DMA,
        ],
    )
    def sck(table_hbm, idx_hbm, out_hbm, idx_v, rows_v, sem):
        wid = lax.axis_index("s") * 2 + lax.axis_index("c")
        base0 = wid * per_w

        def chunk(i, carry):
            base = base0 + i * CH
            pltpu.sync_copy(idx_hbm.at[pl.ds(base, CH)], idx_v)
            pltpu.async_copy(table_hbm.at[idx_v], rows_v, sem).wait()
            pltpu.sync_copy(rows_v, out_hbm.at[pl.ds(base, CH)])
            return carry

        lax.fori_loop(0, n_chunks, chunk, 0)

    return sck(table, idx)


@functools.partial(jax.jit, static_argnames=())
def kernel(x, points, mask, W01, b01, W02, b02, W11, b11, W12, b12):
    B, N, F = x.shape
    P = W02.shape[1]
    mask_f = mask.astype(jnp.float32)
    b01r = b01.reshape(1, -1)
    b02r = b02.reshape(1, -1)
    b11r = b11.reshape(1, -1)
    b12r = b12.reshape(1, -1)
    full = lambda s: pl.BlockSpec(s, lambda b: (0,) * len(s))
    bs_nf = pl.BlockSpec((1, N, F), lambda b: (b, 0, 0))
    bs_m = pl.BlockSpec((1, N, 1), lambda b: (b, 0, 0))
    bs_idx = pl.BlockSpec((1, K, N), lambda b: (b, 0, 0))
    bs_g = pl.BlockSpec((1, K * N, 128), lambda b: (b, 0, 0))

    idx0 = pl.pallas_call(
        _body_a, grid=(B,),
        in_specs=[pl.BlockSpec((1, N, 3), lambda b: (b, 0, 0)), bs_m],
        out_specs=bs_idx,
        out_shape=jax.ShapeDtypeStruct((B, K, N), jnp.int32),
    )(points, mask_f)

    xp = jnp.concatenate(
        [x.reshape(B * N, F),
         jnp.zeros((B * N, 128 - F), jnp.float32)], axis=1)
    g0 = _sc_gather(xp, idx0.reshape(B * K * N))

    f1, idx1 = pl.pallas_call(
        _body_b, grid=(B,),
        in_specs=[bs_nf, bs_g, bs_m,
                  full(W01.shape), full(b01r.shape),
                  full(W02.shape), full(b02r.shape)],
        out_specs=[bs_nf, bs_idx],
        out_shape=[jax.ShapeDtypeStruct((B, N, F), jnp.float32),
                   jax.ShapeDtypeStruct((B, K, N), jnp.int32)],
    )(x, g0.reshape(B, K * N, 128), mask_f, W01, b01r, W02, b02r)

    f1p = jnp.concatenate(
        [f1.reshape(B * N, F),
         jnp.zeros((B * N, 128 - F), jnp.float32)], axis=1)
    g1 = _sc_gather(f1p, idx1.reshape(B * K * N))

    out = pl.pallas_call(
        _body_c, grid=(B,),
        in_specs=[bs_nf, bs_g, bs_m,
                  full(W11.shape), full(b11r.shape),
                  full(W12.shape), full(b12r.shape)],
        out_specs=pl.BlockSpec((1, N, P), lambda b: (b, 0, 0)),
        out_shape=jax.ShapeDtypeStruct((B, N, P), jnp.float32),
    )(f1, g1.reshape(B, K * N, 128), mask_f, W11, b11r, W12, b12r)
    return out


# speedup vs baseline: 2.0008x; 2.0008x over previous
import functools

import jax
import jax.numpy as jnp
from jax import lax
from jax.experimental import pallas as pl

K = 16
NEG_INF = float("-inf")


def _gelu(v):
    return 0.5 * v * (1.0 + lax.erf(v * 0.7071067811865476))


def _local_block(p, f, W1, b1, W2, b2, iota_l, mlp_prec):
    N = p.shape[0]
    F = f.shape[1]
    HI = lax.Precision.HIGHEST
    pp = p * p
    rcol = jnp.sum(pp, axis=1, keepdims=True)
    rrow = jnp.reshape(jnp.sum(pp, axis=1), (1, N))
    m = lax.dot_general(p, p, (((1,), (1,)), ((), ())),
                        precision=HI)
    negD = -(rcol - 2.0 * m + rrow + 1e-05)

    nd = negD
    ohs = []
    for _ in range(K + 1):
        mx = jnp.max(nd, axis=1, keepdims=True)
        ohb = nd == mx
        ohs.append(ohb)
        nd = jnp.where(ohb, NEG_INF, nd)

    W1a = W1[:F, :]
    W1d = W1[F:, :] - W1a
    cterm = lax.dot_general(f, W1d, (((1,), (0,)), ((), ())),
                            precision=mlp_prec) + b1

    if mlp_prec is lax.Precision.HIGHEST:
        fa = f.astype(jnp.bfloat16).astype(jnp.float32)
        fr = f - fa
        fb = fr.astype(jnp.bfloat16).astype(jnp.float32)
        fc = fr - fb
        fparts = (fa, fb, fc)
    else:
        fparts = (f,)

    acc = None
    for k in range(1, K + 1):
        oh = ohs[k].astype(jnp.float32)
        g = None
        for part in fparts:
            gp = lax.dot_general(oh, part, (((1,), (0,)), ((), ())),
                                 precision=lax.Precision.DEFAULT)
            g = gp if g is None else g + gp
        h1 = _gelu(lax.dot_general(g, W1a, (((1,), (0,)), ((), ())),
                                   precision=mlp_prec) + cterm)
        h2 = _gelu(lax.dot_general(h1, W2, (((1,), (0,)), ((), ())),
                                   precision=mlp_prec) + b2)
        acc = h2 if acc is None else acc + h2
    return acc * (1.0 / K)


MB = 1


def _body(x_ref, pts_ref, mask_ref, W01_ref, b01_ref, W02_ref, b02_ref,
          W11_ref, b11_ref, W12_ref, b12_ref, out_ref):
    N = x_ref.shape[1]
    iota_l = lax.broadcasted_iota(jnp.int32, (N, N), 1)
    HI = lax.Precision.HIGHEST
    for i in range(MB):
        f = x_ref[i]
        pts = pts_ref[i]
        mcol = mask_ref[i]
        shift = 999.0 * (mcol == 0.0).astype(jnp.float32)
        f1 = _local_block(shift + pts, f, W01_ref[...], b01_ref[...],
                          W02_ref[...], b02_ref[...], iota_l, HI)
        f2 = _local_block(shift + f1, f1, W11_ref[...], b11_ref[...],
                          W12_ref[...], b12_ref[...], iota_l,
                          lax.Precision.DEFAULT)
        out_ref[i] = f2 * mcol


@functools.partial(jax.jit, static_argnames=())
def kernel(x, points, mask, W01, b01, W02, b02, W11, b11, W12, b12):
    B, N, F = x.shape
    P = W02.shape[1]
    mask_f = mask.astype(jnp.float32)
    b01r = b01.reshape(1, -1)
    b02r = b02.reshape(1, -1)
    b11r = b11.reshape(1, -1)
    b12r = b12.reshape(1, -1)

    full = lambda s: pl.BlockSpec(s, lambda b: (0,) * len(s))
    out = pl.pallas_call(
        _body,
        grid=(B // MB,),
        in_specs=[
            pl.BlockSpec((MB, N, F), lambda b: (b, 0, 0)),
            pl.BlockSpec((MB, N, 3), lambda b: (b, 0, 0)),
            pl.BlockSpec((MB, N, 1), lambda b: (b, 0, 0)),
            full(W01.shape), full(b01r.shape), full(W02.shape), full(b02r.shape),
            full(W11.shape), full(b11r.shape), full(W12.shape), full(b12r.shape),
        ],
        out_specs=pl.BlockSpec((MB, N, P), lambda b: (b, 0, 0)),
        out_shape=jax.ShapeDtypeStruct((B, N, P), jnp.float32),
    )(x, points, mask_f, W01, b01r, W02, b02r, W11, b11r, W12, b12r)
    return out


# speedup vs baseline: 2.0395x; 1.0193x over previous
import functools

import jax
import jax.numpy as jnp
from jax import lax
from jax.experimental import pallas as pl

K = 16
NEG_INF = float("-inf")


def _gelu(v):
    return 0.5 * v * (1.0 + lax.erf(v * 0.7071067811865476))


def _local_block(p, f, W1, b1, W2, b2, iota_l, mlp_prec):
    N = p.shape[0]
    F = f.shape[1]
    HI = lax.Precision.HIGHEST
    pp = p * p
    rcol = jnp.sum(pp, axis=1, keepdims=True)
    rrow = jnp.reshape(jnp.sum(pp, axis=1), (1, N))
    m = lax.dot_general(p, p, (((1,), (1,)), ((), ())),
                        precision=HI)
    negD = -(rcol - 2.0 * m + rrow + 1e-05)

    nd = negD
    ohs = []
    for _ in range(K + 1):
        mx = jnp.max(nd, axis=1, keepdims=True)
        ohb = nd == mx
        ohs.append(ohb)
        nd = jnp.where(ohb, NEG_INF, nd)

    W1a = W1[:F, :]
    W1d = W1[F:, :] - W1a
    cterm = lax.dot_general(f, W1d, (((1,), (0,)), ((), ())),
                            precision=mlp_prec) + b1

    if mlp_prec is lax.Precision.HIGHEST:
        fa = f.astype(jnp.bfloat16).astype(jnp.float32)
        fr = f - fa
        fb = fr.astype(jnp.bfloat16).astype(jnp.float32)
        fc = fr - fb
        fparts = (fa, fb, fc)
    else:
        fparts = (f,)

    acc = None
    for k in range(1, K + 1):
        oh = ohs[k].astype(jnp.float32)
        g = None
        for part in fparts:
            gp = lax.dot_general(oh, part, (((1,), (0,)), ((), ())),
                                 precision=lax.Precision.DEFAULT)
            g = gp if g is None else g + gp
        h1 = _gelu(lax.dot_general(g, W1a, (((1,), (0,)), ((), ())),
                                   precision=mlp_prec) + cterm)
        h2 = _gelu(lax.dot_general(h1, W2, (((1,), (0,)), ((), ())),
                                   precision=mlp_prec) + b2)
        acc = h2 if acc is None else acc + h2
    return acc * (1.0 / K)


MB = 2


def _body(x_ref, pts_ref, mask_ref, W01_ref, b01_ref, W02_ref, b02_ref,
          W11_ref, b11_ref, W12_ref, b12_ref, out_ref):
    N = x_ref.shape[1]
    iota_l = lax.broadcasted_iota(jnp.int32, (N, N), 1)
    HI = lax.Precision.HIGHEST
    for i in range(MB):
        f = x_ref[i]
        pts = pts_ref[i]
        mcol = mask_ref[i]
        shift = 999.0 * (mcol == 0.0).astype(jnp.float32)
        f1 = _local_block(shift + pts, f, W01_ref[...], b01_ref[...],
                          W02_ref[...], b02_ref[...], iota_l, HI)
        f2 = _local_block(shift + f1, f1, W11_ref[...], b11_ref[...],
                          W12_ref[...], b12_ref[...], iota_l,
                          lax.Precision.DEFAULT)
        out_ref[i] = f2 * mcol


@functools.partial(jax.jit, static_argnames=())
def kernel(x, points, mask, W01, b01, W02, b02, W11, b11, W12, b12):
    B, N, F = x.shape
    P = W02.shape[1]
    mask_f = mask.astype(jnp.float32)
    b01r = b01.reshape(1, -1)
    b02r = b02.reshape(1, -1)
    b11r = b11.reshape(1, -1)
    b12r = b12.reshape(1, -1)

    full = lambda s: pl.BlockSpec(s, lambda b: (0,) * len(s))
    out = pl.pallas_call(
        _body,
        grid=(B // MB,),
        in_specs=[
            pl.BlockSpec((MB, N, F), lambda b: (b, 0, 0)),
            pl.BlockSpec((MB, N, 3), lambda b: (b, 0, 0)),
            pl.BlockSpec((MB, N, 1), lambda b: (b, 0, 0)),
            full(W01.shape), full(b01r.shape), full(W02.shape), full(b02r.shape),
            full(W11.shape), full(b11r.shape), full(W12.shape), full(b12r.shape),
        ],
        out_specs=pl.BlockSpec((MB, N, P), lambda b: (b, 0, 0)),
        out_shape=jax.ShapeDtypeStruct((B, N, P), jnp.float32),
    )(x, points, mask_f, W01, b01r, W02, b02r, W11, b11r, W12, b12r)
    return out
